# Initial kernel scaffold; baseline (speedup 1.0000x reference)
import jax
import jax.numpy as jnp
from jax.experimental import pallas as pl


def kernel(feature, edge_index, embedding, conv_W, mlp_W):
    raise NotImplementedError("write your pallas kernel here")


# speedup vs baseline: 16.2670x; 16.2670x over previous
import functools

import jax
import jax.numpy as jnp
from jax import lax
from jax.experimental import pallas as pl
from jax.experimental.pallas import tpu as pltpu
from jax.experimental.pallas import tpu_sc as plsc

N = 10000
E = 320000
D = 128
NC = 2
NS = 16
NW = NC * NS

NPAD = 10240
RPW = NPAD // NS
CH = 128
EPW = 10112
EPAD = EPW * NW
NCHUNK = EPW // CH

_mesh = plsc.VectorSubcoreMesh(
    core_axis_name="c", subcore_axis_name="s", num_cores=NC, num_subcores=NS)
_sc_params = pltpu.CompilerParams(needs_layout_passes=False)


def _deg_body(dst_hbm, deg_out, idxbuf, hist, vbuf, tot, shared):
    c = lax.axis_index("c")
    s = lax.axis_index("s")
    wid = s * NC + c

    def _zero(i, _):
        hist[pl.ds(i * 16, 16)] = jnp.zeros((16,), jnp.float32)
        return _
    lax.fori_loop(0, NPAD // 16, _zero, None)

    pltpu.sync_copy(dst_hbm.at[pl.ds(wid * EPW, EPW)], idxbuf)

    ones = jnp.ones((16,), jnp.float32)

    def _hist(i, _):
        idx = idxbuf[pl.ds(i * 16, 16)]
        plsc.addupdate_scatter(hist, [idx], ones)
        return _
    lax.fori_loop(0, EPW // 16, _hist, None)

    pltpu.sync_copy(hist, shared.at[s])
    plsc.subcore_barrier()

    for r in range(NS):
        pltpu.sync_copy(shared.at[r, pl.ds(s * RPW, RPW)], vbuf.at[r])

    def _comb(v, _):
        a = vbuf[0, pl.ds(v * 16, 16)]
        for r in range(1, NS):
            a = a + vbuf[r, pl.ds(v * 16, 16)]
        tot[pl.ds(v * 16, 16)] = a
        return _
    lax.fori_loop(0, RPW // 16, _comb, None)

    pltpu.sync_copy(tot, deg_out.at[c, pl.ds(s * RPW, RPW)])


_deg_call = functools.partial(
    pl.kernel,
    out_type=jax.ShapeDtypeStruct((NC, NPAD), jnp.float32),
    mesh=_mesh,
    scratch_types=[
        pltpu.VMEM((EPW,), jnp.int32),
        pltpu.VMEM((NPAD,), jnp.float32),
        pltpu.VMEM((NS, RPW), jnp.float32),
        pltpu.VMEM((RPW,), jnp.float32),
        pltpu.VMEM_SHARED((NS, NPAD), jnp.float32),
    ],
    compiler_params=_sc_params,
)(_deg_body)


def _dense_body(emb, feat, wc, wm, d0, d1, g_o, mlp_o, dinv_o):
    dv = lax.rsqrt(d0[...] + d1[...] + 1.0)
    h = lax.dot_general(emb[...], wc[...], (((1,), (1,)), ((), ())),
                        preferred_element_type=jnp.float32)
    g_o[...] = h * dv
    mlp_o[...] = lax.dot_general(feat[...], wm[...], (((1,), (1,)), ((), ())),
                                 preferred_element_type=jnp.float32)
    dinv_o[...] = dv


_RB = 256


def _dense_call(emb_p, feat_p, conv_W, mlp_W, d0, d1):
    grid = (NPAD // _RB,)
    return pl.pallas_call(
        _dense_body,
        grid=grid,
        in_specs=[
            pl.BlockSpec((_RB, D), lambda i: (i, 0)),
            pl.BlockSpec((_RB, D), lambda i: (i, 0)),
            pl.BlockSpec((D, D), lambda i: (0, 0)),
            pl.BlockSpec((D, D), lambda i: (0, 0)),
            pl.BlockSpec((_RB, 1), lambda i: (i, 0)),
            pl.BlockSpec((_RB, 1), lambda i: (i, 0)),
        ],
        out_specs=[
            pl.BlockSpec((_RB, D), lambda i: (i, 0)),
            pl.BlockSpec((_RB, D), lambda i: (i, 0)),
            pl.BlockSpec((_RB, 1), lambda i: (i, 0)),
        ],
        out_shape=[
            jax.ShapeDtypeStruct((NPAD, D), jnp.float32),
            jax.ShapeDtypeStruct((NPAD, D), jnp.float32),
            jax.ShapeDtypeStruct((NPAD, 1), jnp.float32),
        ],
    )(emb_p, feat_p, conv_W, mlp_W, d0, d1)


def _mp_body(src_hbm, dst_hbm, g_hbm, out_hbm, idx_s, idx_d, rows, zbuf, sem,
             acc):
    c = lax.axis_index("c")
    s = lax.axis_index("s")
    wid = s * NC + c

    def _zrow(r, _):
        for k in range(D // 16):
            zbuf[r, pl.ds(k * 16, 16)] = jnp.zeros((16,), jnp.float32)
        return _
    lax.fori_loop(0, 64, _zrow, None)
    for j in range(RPW // 64):
        pltpu.sync_copy(zbuf, acc.at[pl.ds(s * RPW + j * 64, 64)])
    plsc.subcore_barrier()

    def _edge(t, _):
        base = wid * EPW + t * CH
        pltpu.sync_copy(src_hbm.at[pl.ds(base, CH)], idx_s)
        pltpu.sync_copy(dst_hbm.at[pl.ds(base, CH)], idx_d)
        pltpu.async_copy(g_hbm.at[idx_s], rows, sem).wait()
        pltpu.sync_copy(rows, acc.at[idx_d], add=True)
        return _
    lax.fori_loop(0, NCHUNK, _edge, None)

    plsc.subcore_barrier()
    pltpu.sync_copy(acc.at[pl.ds(s * RPW, RPW)],
                    out_hbm.at[c, pl.ds(s * RPW, RPW)])


_mp_call = functools.partial(
    pl.kernel,
    out_type=jax.ShapeDtypeStruct((NC, NPAD, D), jnp.float32),
    mesh=_mesh,
    scratch_types=[
        pltpu.VMEM((CH,), jnp.int32),
        pltpu.VMEM((CH,), jnp.int32),
        pltpu.VMEM((CH, D), jnp.float32),
        pltpu.VMEM((64, D), jnp.float32),
        pltpu.SemaphoreType.DMA,
        pltpu.VMEM_SHARED((NPAD, D), jnp.float32),
    ],
    compiler_params=_sc_params,
)(_mp_body)


def _comb_body(a0, a1, g, mlp, dinv, o):
    o[...] = jnp.maximum(
        (a0[...] + a1[...] + g[...]) * dinv[...] + mlp[...], 0.0)


def _comb_call(a0, a1, g, mlp, dinv):
    grid = (NPAD // _RB,)
    return pl.pallas_call(
        _comb_body,
        grid=grid,
        in_specs=[
            pl.BlockSpec((_RB, D), lambda i: (i, 0)),
            pl.BlockSpec((_RB, D), lambda i: (i, 0)),
            pl.BlockSpec((_RB, D), lambda i: (i, 0)),
            pl.BlockSpec((_RB, D), lambda i: (i, 0)),
            pl.BlockSpec((_RB, 1), lambda i: (i, 0)),
        ],
        out_specs=pl.BlockSpec((_RB, D), lambda i: (i, 0)),
        out_shape=jax.ShapeDtypeStruct((N, D), jnp.float32),
    )(a0, a1, g, mlp, dinv)


def kernel(feature, edge_index, embedding, conv_W, mlp_W):
    src = edge_index[0].astype(jnp.int32)
    dst = edge_index[1].astype(jnp.int32)
    pad = jnp.full((EPAD - E,), N, jnp.int32)
    src_p = jnp.concatenate([src, pad])
    dst_p = jnp.concatenate([dst, pad])
    emb_p = jnp.pad(embedding, ((0, NPAD - N), (0, 0)))
    feat_p = jnp.pad(feature, ((0, NPAD - N), (0, 0)))

    deg2 = _deg_call(dst_p)
    d0 = deg2[0][:, None]
    d1 = deg2[1][:, None]
    g, mlp, dinv = _dense_call(emb_p, feat_p, conv_W, mlp_W, d0, d1)
    accs = _mp_call(src_p, dst_p, g)
    return _comb_call(accs[0], accs[1], g, mlp, dinv)


# speedup vs baseline: 17.4598x; 1.0733x over previous
import functools

import jax
import jax.numpy as jnp
from jax import lax
from jax.experimental import pallas as pl
from jax.experimental.pallas import tpu as pltpu
from jax.experimental.pallas import tpu_sc as plsc

N = 10000
E = 320000
D = 128
DH = D // 2
NC = 2
NS = 16
NW = NC * NS

NPAD = 10240
RPW = NPAD // NS
CH = 128
TCH = 160
EPAD = NS * TCH * CH
NBUF = 4

_mesh = plsc.VectorSubcoreMesh(
    core_axis_name="c", subcore_axis_name="s", num_cores=NC, num_subcores=NS)
_sc_params = pltpu.CompilerParams(needs_layout_passes=False,
                                  use_tc_tiling_on_sc=False)


def _deg_body(dst_hbm, deg_out, idxbuf, hist, vbuf, tot, shared):
    c = lax.axis_index("c")
    s = lax.axis_index("s")
    wid = s * NC + c
    npc = EPAD // NW // CH

    def _zero(i, _):
        hist[pl.ds(i * 16, 16)] = jnp.zeros((16,), jnp.float32)
        return _
    lax.fori_loop(0, NPAD // 16, _zero, None)

    pltpu.sync_copy(dst_hbm.at[pl.ds(wid * npc, npc)], idxbuf)

    ones = jnp.ones((16,), jnp.float32)

    def _hist(t, _):
        for u in range(CH // 16):
            idx = idxbuf[t, pl.ds(u * 16, 16)]
            plsc.addupdate_scatter(hist, [idx], ones)
        return _
    lax.fori_loop(0, npc, _hist, None)

    pltpu.sync_copy(hist, shared.at[s])
    plsc.subcore_barrier()

    for r in range(NS):
        pltpu.sync_copy(shared.at[r, pl.ds(s * RPW, RPW)], vbuf.at[r])

    def _comb(v, _):
        a = vbuf[0, pl.ds(v * 16, 16)]
        for r in range(1, NS):
            a = a + vbuf[r, pl.ds(v * 16, 16)]
        tot[pl.ds(v * 16, 16)] = a
        return _
    lax.fori_loop(0, RPW // 16, _comb, None)

    pltpu.sync_copy(tot, deg_out.at[c, pl.ds(s * RPW, RPW)])


_deg_call = functools.partial(
    pl.kernel,
    out_type=jax.ShapeDtypeStruct((NC, NPAD), jnp.float32),
    mesh=_mesh,
    scratch_types=[
        pltpu.VMEM((EPAD // NW // CH, CH), jnp.int32),
        pltpu.VMEM((NPAD,), jnp.float32),
        pltpu.VMEM((NS, RPW), jnp.float32),
        pltpu.VMEM((RPW,), jnp.float32),
        pltpu.VMEM_SHARED((NS, NPAD), jnp.float32),
    ],
    compiler_params=_sc_params,
)(_deg_body)


def _dense_body(emb, feat, wc, wm, d0, d1, g0_o, g1_o, mlp_o, dinv_o):
    i = pl.program_id(0)
    dv = lax.rsqrt(d0[...] + d1[...] + 1.0)
    h = lax.dot_general(emb[...], wc[...], (((1,), (1,)), ((), ())),
                        preferred_element_type=jnp.float32)
    rid = i * _RB + lax.broadcasted_iota(jnp.int32, (_RB, 1), 0)
    g = jnp.where(rid < N, h * dv, 0.0)
    g0_o[...] = g[:, :DH]
    g1_o[...] = g[:, DH:]
    mlp_o[...] = lax.dot_general(feat[...], wm[...], (((1,), (1,)), ((), ())),
                                 preferred_element_type=jnp.float32)
    dinv_o[...] = dv


_RB = 256


def _dense_call(emb, feat, conv_W, mlp_W, d0, d1):
    grid = (NPAD // _RB,)
    return pl.pallas_call(
        _dense_body,
        grid=grid,
        in_specs=[
            pl.BlockSpec((_RB, D), lambda i: (i, 0)),
            pl.BlockSpec((_RB, D), lambda i: (i, 0)),
            pl.BlockSpec((D, D), lambda i: (0, 0)),
            pl.BlockSpec((D, D), lambda i: (0, 0)),
            pl.BlockSpec((_RB, 1), lambda i: (i, 0)),
            pl.BlockSpec((_RB, 1), lambda i: (i, 0)),
        ],
        out_specs=[
            pl.BlockSpec((_RB, DH), lambda i: (i, 0)),
            pl.BlockSpec((_RB, DH), lambda i: (i, 0)),
            pl.BlockSpec((_RB, D), lambda i: (i, 0)),
            pl.BlockSpec((_RB, 1), lambda i: (i, 0)),
        ],
        out_shape=[
            jax.ShapeDtypeStruct((NPAD, DH), jnp.float32),
            jax.ShapeDtypeStruct((NPAD, DH), jnp.float32),
            jax.ShapeDtypeStruct((NPAD, D), jnp.float32),
            jax.ShapeDtypeStruct((NPAD, 1), jnp.float32),
        ],
    )(emb, feat, conv_W, mlp_W, d0, d1)


def _mp_body(src_hbm, dst_hbm, g0_hbm, g1_hbm, out_hbm, idx_s, idx_d, rows,
             zbuf, isem_s, isem_d, gsems, acc):
    c = lax.axis_index("c")
    s = lax.axis_index("s")

    cs = pltpu.async_copy(src_hbm.at[pl.ds(s * TCH, TCH)], idx_s, isem_s)
    cd = pltpu.async_copy(dst_hbm.at[pl.ds(s * TCH, TCH)], idx_d, isem_d)

    def _zrow(r, _):
        for k in range(DH // 16):
            zbuf[r, pl.ds(k * 16, 16)] = jnp.zeros((16,), jnp.float32)
        return _
    lax.fori_loop(0, 64, _zrow, None)
    for j in range(RPW // 64):
        pltpu.sync_copy(zbuf, acc.at[pl.ds(s * RPW + j * 64, 64)])
    cs.wait()
    cd.wait()
    plsc.subcore_barrier()

    def _run(g_hbm):
        for b in range(NBUF):
            pltpu.async_copy(g_hbm.at[idx_s.at[b]], rows.at[b], gsems.at[b])

        def _edge(i, _):
            for b in range(NBUF):
                t = i * NBUF + b
                pltpu.make_async_copy(g_hbm.at[idx_s.at[t]], rows.at[b],
                                      gsems.at[b]).wait()
                pltpu.sync_copy(rows.at[b], acc.at[idx_d.at[t]], add=True)
                tn = t + NBUF

                @pl.when(tn < TCH)
                def _():
                    pltpu.async_copy(g_hbm.at[idx_s.at[tn]], rows.at[b],
                                     gsems.at[b])
            return _
        lax.fori_loop(0, TCH // NBUF, _edge, None)

    @pl.when(c == 0)
    def _():
        _run(g0_hbm)

    @pl.when(c == 1)
    def _():
        _run(g1_hbm)

    plsc.subcore_barrier()
    pltpu.sync_copy(acc.at[pl.ds(s * RPW, RPW)],
                    out_hbm.at[c, pl.ds(s * RPW, RPW)])


_mp_call = functools.partial(
    pl.kernel,
    out_type=jax.ShapeDtypeStruct((NC, NPAD, DH), jnp.float32),
    mesh=_mesh,
    scratch_types=[
        pltpu.VMEM((TCH, CH), jnp.int32),
        pltpu.VMEM((TCH, CH), jnp.int32),
        pltpu.VMEM((NBUF, CH, DH), jnp.float32),
        pltpu.VMEM((64, DH), jnp.float32),
        pltpu.SemaphoreType.DMA,
        pltpu.SemaphoreType.DMA,
        pltpu.SemaphoreType.DMA((NBUF,)),
        pltpu.VMEM_SHARED((NPAD, DH), jnp.float32),
    ],
    compiler_params=_sc_params,
)(_mp_body)


def _comb_body(a0, a1, g0, g1, mlp, dinv, o):
    dv = dinv[...]
    m = mlp[...]
    left = (a0[...] + g0[...]) * dv + m[:, :DH]
    right = (a1[...] + g1[...]) * dv + m[:, DH:]
    o[...] = jnp.maximum(jnp.concatenate([left, right], axis=1), 0.0)


def _comb_call(a0, a1, g0, g1, mlp, dinv):
    grid = (NPAD // _RB,)
    return pl.pallas_call(
        _comb_body,
        grid=grid,
        in_specs=[
            pl.BlockSpec((_RB, DH), lambda i: (i, 0)),
            pl.BlockSpec((_RB, DH), lambda i: (i, 0)),
            pl.BlockSpec((_RB, DH), lambda i: (i, 0)),
            pl.BlockSpec((_RB, DH), lambda i: (i, 0)),
            pl.BlockSpec((_RB, D), lambda i: (i, 0)),
            pl.BlockSpec((_RB, 1), lambda i: (i, 0)),
        ],
        out_specs=pl.BlockSpec((_RB, D), lambda i: (i, 0)),
        out_shape=jax.ShapeDtypeStruct((N, D), jnp.float32),
    )(a0, a1, g0, g1, mlp, dinv)


def kernel(feature, edge_index, embedding, conv_W, mlp_W):
    src = edge_index[0].astype(jnp.int32)
    dst = edge_index[1].astype(jnp.int32)
    pad = jnp.full((EPAD - E,), N, jnp.int32)
    src_p = jnp.concatenate([src, pad]).reshape(NS * TCH, CH)
    dst_p = jnp.concatenate([dst, pad]).reshape(NS * TCH, CH)

    deg2 = _deg_call(dst_p)
    d0 = deg2[0][:, None]
    d1 = deg2[1][:, None]
    g0, g1, mlp, dinv = _dense_call(embedding, feature, conv_W, mlp_W, d0, d1)
    accs = _mp_call(src_p, dst_p, g0, g1)
    return _comb_call(accs[0], accs[1], g0, g1, mlp, dinv)


# speedup vs baseline: 25.6025x; 1.4664x over previous
import functools

import jax
import jax.numpy as jnp
from jax import lax
from jax.experimental import pallas as pl
from jax.experimental.pallas import tpu as pltpu
from jax.experimental.pallas import tpu_sc as plsc

N = 10000
E = 320000
D = 128
DH = D // 2
NC = 2
NS = 16
NW = NC * NS

NPAD = 10240
RPW = NPAD // NS
CH = 128
TCH = 160
EPAD = NS * TCH * CH
NBUF = 4

_mesh = plsc.VectorSubcoreMesh(
    core_axis_name="c", subcore_axis_name="s", num_cores=NC, num_subcores=NS)
_sc_params = pltpu.CompilerParams(needs_layout_passes=False,
                                  use_tc_tiling_on_sc=False,
                                  internal_scratch_in_bytes=128 * 1024)


def _deg_body(dst_hbm, deg_out, idxbuf, hist, vbuf, tot, shared):
    c = lax.axis_index("c")
    s = lax.axis_index("s")
    wid = s * NC + c
    npc = EPAD // NW // CH

    def _zero(i, _):
        hist[pl.ds(i * 16, 16)] = jnp.zeros((16,), jnp.float32)
        return _
    lax.fori_loop(0, NPAD // 16, _zero, None)

    pltpu.sync_copy(dst_hbm.at[pl.ds(wid * npc, npc)], idxbuf)

    ones = jnp.ones((16,), jnp.float32)

    def _hist(t, _):
        for u in range(CH // 16):
            idx = idxbuf[t, pl.ds(u * 16, 16)]
            plsc.addupdate_scatter(hist, [idx], ones)
        return _
    lax.fori_loop(0, npc, _hist, None)

    pltpu.sync_copy(hist, shared.at[s])
    plsc.subcore_barrier()

    for r in range(NS):
        pltpu.sync_copy(shared.at[r, pl.ds(s * RPW, RPW)], vbuf.at[r])

    def _comb(v, _):
        a = vbuf[0, pl.ds(v * 16, 16)]
        for r in range(1, NS):
            a = a + vbuf[r, pl.ds(v * 16, 16)]
        tot[pl.ds(v * 16, 16)] = a
        return _
    lax.fori_loop(0, RPW // 16, _comb, None)

    pltpu.sync_copy(tot, deg_out.at[c, pl.ds(s * RPW, RPW)])


_deg_call = functools.partial(
    pl.kernel,
    out_type=jax.ShapeDtypeStruct((NC, NPAD), jnp.float32),
    mesh=_mesh,
    scratch_types=[
        pltpu.VMEM((EPAD // NW // CH, CH), jnp.int32),
        pltpu.VMEM((NPAD,), jnp.float32),
        pltpu.VMEM((NS, RPW), jnp.float32),
        pltpu.VMEM((RPW,), jnp.float32),
        pltpu.VMEM_SHARED((NS, NPAD), jnp.float32),
    ],
    compiler_params=_sc_params,
)(_deg_body)


def _dense_body(emb, feat, wc, wm, d0, d1, g0_o, g1_o, mlp_o, dinv_o):
    i = pl.program_id(0)
    dv = lax.rsqrt(d0[...] + d1[...] + 1.0)
    h = lax.dot_general(emb[...], wc[...], (((1,), (1,)), ((), ())),
                        preferred_element_type=jnp.float32)
    rid = i * _RB + lax.broadcasted_iota(jnp.int32, (_RB, 1), 0)
    g = jnp.where(rid < N, h * dv, 0.0)
    g0_o[...] = g[:, :DH]
    g1_o[...] = g[:, DH:]
    mlp_o[...] = lax.dot_general(feat[...], wm[...], (((1,), (1,)), ((), ())),
                                 preferred_element_type=jnp.float32)
    dinv_o[...] = dv


_RB = 256


def _dense_call(emb, feat, conv_W, mlp_W, d0, d1):
    grid = (NPAD // _RB,)
    return pl.pallas_call(
        _dense_body,
        grid=grid,
        in_specs=[
            pl.BlockSpec((_RB, D), lambda i: (i, 0)),
            pl.BlockSpec((_RB, D), lambda i: (i, 0)),
            pl.BlockSpec((D, D), lambda i: (0, 0)),
            pl.BlockSpec((D, D), lambda i: (0, 0)),
            pl.BlockSpec((_RB, 1), lambda i: (i, 0)),
            pl.BlockSpec((_RB, 1), lambda i: (i, 0)),
        ],
        out_specs=[
            pl.BlockSpec((_RB, DH), lambda i: (i, 0)),
            pl.BlockSpec((_RB, DH), lambda i: (i, 0)),
            pl.BlockSpec((_RB, D), lambda i: (i, 0)),
            pl.BlockSpec((_RB, 1), lambda i: (i, 0)),
        ],
        out_shape=[
            jax.ShapeDtypeStruct((NPAD, DH), jnp.float32),
            jax.ShapeDtypeStruct((NPAD, DH), jnp.float32),
            jax.ShapeDtypeStruct((NPAD, D), jnp.float32),
            jax.ShapeDtypeStruct((NPAD, 1), jnp.float32),
        ],
    )(emb, feat, conv_W, mlp_W, d0, d1)


IBLK = 20
NPAIR = TCH // (2 * IBLK)


def _mp_body(src_hbm, dst_hbm, g0_hbm, g1_hbm, out_hbm, idxs0, idxd0, idxs1,
             idxd1, rows, ls0, ld0, ls1, ld1, gsems, acc, gtab):
    c = lax.axis_index("c")
    s = lax.axis_index("s")

    def _load_blk(k, idxs_b, idxd_b, sem_s, sem_d):
        base = s * TCH + k * IBLK
        pltpu.async_copy(src_hbm.at[pl.ds(base, IBLK)], idxs_b, sem_s)
        pltpu.async_copy(dst_hbm.at[pl.ds(base, IBLK)], idxd_b, sem_d)

    def _wait_blk(k, idxs_b, idxd_b, sem_s, sem_d):
        base = s * TCH + k * IBLK
        pltpu.make_async_copy(src_hbm.at[pl.ds(base, IBLK)], idxs_b,
                              sem_s).wait()
        pltpu.make_async_copy(dst_hbm.at[pl.ds(base, IBLK)], idxd_b,
                              sem_d).wait()

    _load_blk(0, idxs0, idxd0, ls0, ld0)

    @pl.when(c == 0)
    def _():
        pltpu.sync_copy(g0_hbm.at[pl.ds(s * RPW, RPW)],
                        gtab.at[pl.ds(s * RPW, RPW)])

    @pl.when(c == 1)
    def _():
        pltpu.sync_copy(g1_hbm.at[pl.ds(s * RPW, RPW)],
                        gtab.at[pl.ds(s * RPW, RPW)])

    def _zrow(r, _):
        for k in range(DH // 16):
            rows[0, r, pl.ds(k * 16, 16)] = jnp.zeros((16,), jnp.float32)
        return _
    lax.fori_loop(0, CH, _zrow, None)
    for j in range(RPW // CH):
        pltpu.sync_copy(rows.at[0], acc.at[pl.ds(s * RPW + j * CH, CH)])
    plsc.subcore_barrier()

    def _block(idxs_b, idxd_b):
        for b in range(NBUF):
            pltpu.async_copy(gtab.at[idxs_b.at[b]], rows.at[b], gsems.at[b])

        def _chunk(i, _):
            for b in range(NBUF):
                j = i * NBUF + b
                pltpu.make_async_copy(gtab.at[idxs_b.at[j]], rows.at[b],
                                      gsems.at[b]).wait()
                pltpu.sync_copy(rows.at[b], acc.at[idxd_b.at[j]], add=True)
                pltpu.async_copy(gtab.at[idxs_b.at[j + NBUF]], rows.at[b],
                                 gsems.at[b])
            return _
        lax.fori_loop(0, (IBLK - NBUF) // NBUF, _chunk, None)

        for b in range(NBUF):
            j = IBLK - NBUF + b
            pltpu.make_async_copy(gtab.at[idxs_b.at[j]], rows.at[b],
                                  gsems.at[b]).wait()
            pltpu.sync_copy(rows.at[b], acc.at[idxd_b.at[j]], add=True)

    def _pair(i, _):
        k0 = 2 * i
        _wait_blk(k0, idxs0, idxd0, ls0, ld0)
        _load_blk(k0 + 1, idxs1, idxd1, ls1, ld1)
        _block(idxs0, idxd0)
        _wait_blk(k0 + 1, idxs1, idxd1, ls1, ld1)

        @pl.when(i < NPAIR - 1)
        def _():
            _load_blk(k0 + 2, idxs0, idxd0, ls0, ld0)

        _block(idxs1, idxd1)
        return _
    lax.fori_loop(0, NPAIR, _pair, None)

    plsc.subcore_barrier()
    pltpu.sync_copy(acc.at[pl.ds(s * RPW, RPW)],
                    out_hbm.at[c, pl.ds(s * RPW, RPW)])


_mp_call = functools.partial(
    pl.kernel,
    out_type=jax.ShapeDtypeStruct((NC, NPAD, DH), jnp.float32),
    mesh=_mesh,
    scratch_types=[
        pltpu.VMEM((IBLK, CH), jnp.int32),
        pltpu.VMEM((IBLK, CH), jnp.int32),
        pltpu.VMEM((IBLK, CH), jnp.int32),
        pltpu.VMEM((IBLK, CH), jnp.int32),
        pltpu.VMEM((NBUF, CH, DH), jnp.float32),
        pltpu.SemaphoreType.DMA,
        pltpu.SemaphoreType.DMA,
        pltpu.SemaphoreType.DMA,
        pltpu.SemaphoreType.DMA,
        pltpu.SemaphoreType.DMA((NBUF,)),
        pltpu.VMEM_SHARED((NPAD, DH), jnp.float32),
        pltpu.VMEM_SHARED((NPAD, DH), jnp.float32),
    ],
    compiler_params=_sc_params,
)(_mp_body)


def _comb_body(a0, a1, g0, g1, mlp, dinv, o):
    dv = dinv[...]
    m = mlp[...]
    left = (a0[...] + g0[...]) * dv + m[:, :DH]
    right = (a1[...] + g1[...]) * dv + m[:, DH:]
    o[...] = jnp.maximum(jnp.concatenate([left, right], axis=1), 0.0)


def _comb_call(a0, a1, g0, g1, mlp, dinv):
    grid = (NPAD // _RB,)
    return pl.pallas_call(
        _comb_body,
        grid=grid,
        in_specs=[
            pl.BlockSpec((_RB, DH), lambda i: (i, 0)),
            pl.BlockSpec((_RB, DH), lambda i: (i, 0)),
            pl.BlockSpec((_RB, DH), lambda i: (i, 0)),
            pl.BlockSpec((_RB, DH), lambda i: (i, 0)),
            pl.BlockSpec((_RB, D), lambda i: (i, 0)),
            pl.BlockSpec((_RB, 1), lambda i: (i, 0)),
        ],
        out_specs=pl.BlockSpec((_RB, D), lambda i: (i, 0)),
        out_shape=jax.ShapeDtypeStruct((N, D), jnp.float32),
    )(a0, a1, g0, g1, mlp, dinv)


def kernel(feature, edge_index, embedding, conv_W, mlp_W):
    src = edge_index[0].astype(jnp.int32)
    dst = edge_index[1].astype(jnp.int32)
    pad = jnp.full((EPAD - E,), N, jnp.int32)
    src_p = jnp.concatenate([src, pad]).reshape(NS * TCH, CH)
    dst_p = jnp.concatenate([dst, pad]).reshape(NS * TCH, CH)

    deg2 = _deg_call(dst_p)
    d0 = deg2[0][:, None]
    d1 = deg2[1][:, None]
    g0, g1, mlp, dinv = _dense_call(embedding, feature, conv_W, mlp_W, d0, d1)
    accs = _mp_call(src_p, dst_p, g0, g1)
    return _comb_call(accs[0], accs[1], g0, g1, mlp, dinv)


# speedup vs baseline: 34.2958x; 1.3395x over previous
import functools

import jax
import jax.numpy as jnp
from jax import lax
from jax.experimental import pallas as pl
from jax.experimental.pallas import tpu as pltpu
from jax.experimental.pallas import tpu_sc as plsc

N = 10000
E = 320000
D = 128
DH = D // 2
NC = 2
NS = 16
NW = NC * NS

NPAD = 10240
RPW = NPAD // NS
CH = 128
TCH = 160
EPAD = NS * TCH * CH
NBUF = 4

_mesh = plsc.VectorSubcoreMesh(
    core_axis_name="c", subcore_axis_name="s", num_cores=NC, num_subcores=NS)
_sc_params = pltpu.CompilerParams(needs_layout_passes=False,
                                  use_tc_tiling_on_sc=False,
                                  internal_scratch_in_bytes=128 * 1024)


def _deg_body(dst_hbm, deg_out, idxbuf, hist, vbuf, tot, shared):
    c = lax.axis_index("c")
    s = lax.axis_index("s")
    wid = s * NC + c
    npc = EPAD // NW // CH

    def _zero(i, _):
        hist[pl.ds(i * 16, 16)] = jnp.zeros((16,), jnp.float32)
        return _
    lax.fori_loop(0, NPAD // 16, _zero, None)

    pltpu.sync_copy(dst_hbm.at[pl.ds(wid * npc, npc)], idxbuf)

    ones = jnp.ones((16,), jnp.float32)

    def _hist(t, _):
        for u in range(CH // 16):
            idx = idxbuf[t, pl.ds(u * 16, 16)]
            plsc.addupdate_scatter(hist, [idx], ones)
        return _
    lax.fori_loop(0, npc, _hist, None)

    pltpu.sync_copy(hist, shared.at[s])
    plsc.subcore_barrier()

    for r in range(NS):
        pltpu.sync_copy(shared.at[r, pl.ds(s * RPW, RPW)], vbuf.at[r])

    def _comb(v, _):
        a = vbuf[0, pl.ds(v * 16, 16)]
        for r in range(1, NS):
            a = a + vbuf[r, pl.ds(v * 16, 16)]
        tot[pl.ds(v * 16, 16)] = a
        return _
    lax.fori_loop(0, RPW // 16, _comb, None)

    pltpu.sync_copy(tot, deg_out.at[c, pl.ds(s * RPW, RPW)])


_deg_call = functools.partial(
    pl.kernel,
    out_type=jax.ShapeDtypeStruct((NC, NPAD), jnp.float32),
    mesh=_mesh,
    scratch_types=[
        pltpu.VMEM((EPAD // NW // CH, CH), jnp.int32),
        pltpu.VMEM((NPAD,), jnp.float32),
        pltpu.VMEM((NS, RPW), jnp.float32),
        pltpu.VMEM((RPW,), jnp.float32),
        pltpu.VMEM_SHARED((NS, NPAD), jnp.float32),
    ],
    compiler_params=_sc_params,
)(_deg_body)


def _dense_body(emb, feat, wc, wm, d0, d1, g0_o, g1_o, mlp_o, dinv_o):
    i = pl.program_id(0)
    dv = lax.rsqrt(d0[...] + d1[...] + 1.0)
    h = lax.dot_general(emb[...], wc[...], (((1,), (1,)), ((), ())),
                        preferred_element_type=jnp.float32)
    rid = i * _RB + lax.broadcasted_iota(jnp.int32, (_RB, 1), 0)
    g = jnp.where(rid < N, h * dv, 0.0).astype(jnp.bfloat16)
    g0_o[...] = g[:, :DH]
    g1_o[...] = g[:, DH:]
    mlp_o[...] = lax.dot_general(feat[...], wm[...], (((1,), (1,)), ((), ())),
                                 preferred_element_type=jnp.float32)
    dinv_o[...] = dv


_RB = 256


def _dense_call(emb, feat, conv_W, mlp_W, d0, d1):
    grid = (NPAD // _RB,)
    return pl.pallas_call(
        _dense_body,
        grid=grid,
        in_specs=[
            pl.BlockSpec((_RB, D), lambda i: (i, 0)),
            pl.BlockSpec((_RB, D), lambda i: (i, 0)),
            pl.BlockSpec((D, D), lambda i: (0, 0)),
            pl.BlockSpec((D, D), lambda i: (0, 0)),
            pl.BlockSpec((_RB, 1), lambda i: (i, 0)),
            pl.BlockSpec((_RB, 1), lambda i: (i, 0)),
        ],
        out_specs=[
            pl.BlockSpec((_RB, DH), lambda i: (i, 0)),
            pl.BlockSpec((_RB, DH), lambda i: (i, 0)),
            pl.BlockSpec((_RB, D), lambda i: (i, 0)),
            pl.BlockSpec((_RB, 1), lambda i: (i, 0)),
        ],
        out_shape=[
            jax.ShapeDtypeStruct((NPAD, DH), jnp.bfloat16),
            jax.ShapeDtypeStruct((NPAD, DH), jnp.bfloat16),
            jax.ShapeDtypeStruct((NPAD, D), jnp.float32),
            jax.ShapeDtypeStruct((NPAD, 1), jnp.float32),
        ],
    )(emb, feat, conv_W, mlp_W, d0, d1)


IBLK = 20
NPAIR = TCH // (2 * IBLK)


def _mp_body(src_hbm, dst_hbm, g0_hbm, g1_hbm, out_hbm, idxs0, idxd0, idxs1,
             idxd1, rows, ls0, ld0, ls1, ld1, gsems, acc, gtab):
    c = lax.axis_index("c")
    s = lax.axis_index("s")

    def _load_blk(k, idxs_b, idxd_b, sem_s, sem_d):
        base = s * TCH + k * IBLK
        pltpu.async_copy(src_hbm.at[pl.ds(base, IBLK)], idxs_b, sem_s)
        pltpu.async_copy(dst_hbm.at[pl.ds(base, IBLK)], idxd_b, sem_d)

    def _wait_blk(k, idxs_b, idxd_b, sem_s, sem_d):
        base = s * TCH + k * IBLK
        pltpu.make_async_copy(src_hbm.at[pl.ds(base, IBLK)], idxs_b,
                              sem_s).wait()
        pltpu.make_async_copy(dst_hbm.at[pl.ds(base, IBLK)], idxd_b,
                              sem_d).wait()

    _load_blk(0, idxs0, idxd0, ls0, ld0)

    @pl.when(c == 0)
    def _():
        pltpu.sync_copy(g0_hbm.at[pl.ds(s * RPW, RPW)],
                        gtab.at[pl.ds(s * RPW, RPW)])

    @pl.when(c == 1)
    def _():
        pltpu.sync_copy(g1_hbm.at[pl.ds(s * RPW, RPW)],
                        gtab.at[pl.ds(s * RPW, RPW)])

    def _zrow(r, _):
        for k in range(DH // 32):
            rows[0, r, pl.ds(k * 32, 32)] = jnp.zeros((32,), jnp.bfloat16)
        return _
    lax.fori_loop(0, CH, _zrow, None)
    for j in range(RPW // CH):
        pltpu.sync_copy(rows.at[0], acc.at[pl.ds(s * RPW + j * CH, CH)])
    plsc.subcore_barrier()

    def _block(idxs_b, idxd_b):
        for b in range(NBUF):
            pltpu.async_copy(gtab.at[idxs_b.at[b]], rows.at[b], gsems.at[b])

        def _chunk(i, _):
            for b in range(NBUF):
                j = i * NBUF + b
                pltpu.make_async_copy(gtab.at[idxs_b.at[j]], rows.at[b],
                                      gsems.at[b]).wait()
                pltpu.sync_copy(rows.at[b], acc.at[idxd_b.at[j]], add=True)
                pltpu.async_copy(gtab.at[idxs_b.at[j + NBUF]], rows.at[b],
                                 gsems.at[b])
            return _
        lax.fori_loop(0, (IBLK - NBUF) // NBUF, _chunk, None)

        for b in range(NBUF):
            j = IBLK - NBUF + b
            pltpu.make_async_copy(gtab.at[idxs_b.at[j]], rows.at[b],
                                  gsems.at[b]).wait()
            pltpu.sync_copy(rows.at[b], acc.at[idxd_b.at[j]], add=True)

    def _pair(i, _):
        k0 = 2 * i
        _wait_blk(k0, idxs0, idxd0, ls0, ld0)
        _load_blk(k0 + 1, idxs1, idxd1, ls1, ld1)
        _block(idxs0, idxd0)
        _wait_blk(k0 + 1, idxs1, idxd1, ls1, ld1)

        @pl.when(i < NPAIR - 1)
        def _():
            _load_blk(k0 + 2, idxs0, idxd0, ls0, ld0)

        _block(idxs1, idxd1)
        return _
    lax.fori_loop(0, NPAIR, _pair, None)

    plsc.subcore_barrier()
    pltpu.sync_copy(acc.at[pl.ds(s * RPW, RPW)],
                    out_hbm.at[c, pl.ds(s * RPW, RPW)])


_mp_call = functools.partial(
    pl.kernel,
    out_type=jax.ShapeDtypeStruct((NC, NPAD, DH), jnp.bfloat16),
    mesh=_mesh,
    scratch_types=[
        pltpu.VMEM((IBLK, CH), jnp.int32),
        pltpu.VMEM((IBLK, CH), jnp.int32),
        pltpu.VMEM((IBLK, CH), jnp.int32),
        pltpu.VMEM((IBLK, CH), jnp.int32),
        pltpu.VMEM((NBUF, CH, DH), jnp.bfloat16),
        pltpu.SemaphoreType.DMA,
        pltpu.SemaphoreType.DMA,
        pltpu.SemaphoreType.DMA,
        pltpu.SemaphoreType.DMA,
        pltpu.SemaphoreType.DMA((NBUF,)),
        pltpu.VMEM_SHARED((NPAD, DH), jnp.bfloat16),
        pltpu.VMEM_SHARED((NPAD, DH), jnp.bfloat16),
    ],
    compiler_params=_sc_params,
)(_mp_body)


def _comb_body(a0, a1, g0, g1, mlp, dinv, o):
    dv = dinv[...]
    m = mlp[...]
    f32 = jnp.float32
    left = (a0[...].astype(f32) + g0[...].astype(f32)) * dv + m[:, :DH]
    right = (a1[...].astype(f32) + g1[...].astype(f32)) * dv + m[:, DH:]
    o[...] = jnp.maximum(jnp.concatenate([left, right], axis=1), 0.0)


def _comb_call(a0, a1, g0, g1, mlp, dinv):
    grid = (NPAD // _RB,)
    return pl.pallas_call(
        _comb_body,
        grid=grid,
        in_specs=[
            pl.BlockSpec((_RB, DH), lambda i: (i, 0)),
            pl.BlockSpec((_RB, DH), lambda i: (i, 0)),
            pl.BlockSpec((_RB, DH), lambda i: (i, 0)),
            pl.BlockSpec((_RB, DH), lambda i: (i, 0)),
            pl.BlockSpec((_RB, D), lambda i: (i, 0)),
            pl.BlockSpec((_RB, 1), lambda i: (i, 0)),
        ],
        out_specs=pl.BlockSpec((_RB, D), lambda i: (i, 0)),
        out_shape=jax.ShapeDtypeStruct((N, D), jnp.float32),
    )(a0, a1, g0, g1, mlp, dinv)


def kernel(feature, edge_index, embedding, conv_W, mlp_W):
    src = edge_index[0].astype(jnp.int32)
    dst = edge_index[1].astype(jnp.int32)
    pad = jnp.full((EPAD - E,), N, jnp.int32)
    src_p = jnp.concatenate([src, pad]).reshape(NS * TCH, CH)
    dst_p = jnp.concatenate([dst, pad]).reshape(NS * TCH, CH)

    deg2 = _deg_call(dst_p)
    d0 = deg2[0][:, None]
    d1 = deg2[1][:, None]
    g0, g1, mlp, dinv = _dense_call(embedding, feature, conv_W, mlp_W, d0, d1)
    accs = _mp_call(src_p, dst_p, g0, g1)
    return _comb_call(accs[0], accs[1], g0, g1, mlp, dinv)


# speedup vs baseline: 38.0379x; 1.1091x over previous
import functools

import jax
import jax.numpy as jnp
from jax import lax
from jax.experimental import pallas as pl
from jax.experimental.pallas import tpu as pltpu
from jax.experimental.pallas import tpu_sc as plsc

N = 10000
E = 320000
D = 128
DH = D // 2
NC = 2
NS = 16
NW = NC * NS

NPAD = 10240
RPW = NPAD // NS
CH = 128
TCH = 160
EPAD = NS * TCH * CH
NBUF = 4

_mesh = plsc.VectorSubcoreMesh(
    core_axis_name="c", subcore_axis_name="s", num_cores=NC, num_subcores=NS)
_sc_params = pltpu.CompilerParams(needs_layout_passes=False,
                                  use_tc_tiling_on_sc=False,
                                  internal_scratch_in_bytes=128 * 1024)


def _deg_body(dst_hbm, deg_out, idxbuf, hist, vbuf, tot, shared):
    c = lax.axis_index("c")
    s = lax.axis_index("s")
    wid = s * NC + c
    npc = EPAD // NW // CH

    def _zero(i, _):
        hist[pl.ds(i * 16, 16)] = jnp.zeros((16,), jnp.float32)
        return _
    lax.fori_loop(0, NPAD // 16, _zero, None)

    pltpu.sync_copy(dst_hbm.at[pl.ds(wid * npc, npc)], idxbuf)

    ones = jnp.ones((16,), jnp.float32)

    def _hist(t, _):
        for u in range(CH // 16):
            idx = idxbuf[t, pl.ds(u * 16, 16)]
            plsc.addupdate_scatter(hist, [idx], ones)
        return _
    lax.fori_loop(0, npc, _hist, None)

    pltpu.sync_copy(hist, shared.at[s])
    plsc.subcore_barrier()

    for r in range(NS):
        pltpu.sync_copy(shared.at[r, pl.ds(s * RPW, RPW)], vbuf.at[r])

    def _comb(v, _):
        a = vbuf[0, pl.ds(v * 16, 16)]
        for r in range(1, NS):
            a = a + vbuf[r, pl.ds(v * 16, 16)]
        tot[pl.ds(v * 16, 16)] = a
        return _
    lax.fori_loop(0, RPW // 16, _comb, None)

    pltpu.sync_copy(tot, deg_out.at[c, pl.ds(s * RPW, RPW)])


_deg_call = functools.partial(
    pl.kernel,
    out_type=jax.ShapeDtypeStruct((NC, NPAD), jnp.float32),
    mesh=_mesh,
    scratch_types=[
        pltpu.VMEM((EPAD // NW // CH, CH), jnp.int32),
        pltpu.VMEM((NPAD,), jnp.float32),
        pltpu.VMEM((NS, RPW), jnp.float32),
        pltpu.VMEM((RPW,), jnp.float32),
        pltpu.VMEM_SHARED((NS, NPAD), jnp.float32),
    ],
    compiler_params=_sc_params,
)(_deg_body)


def _dense_body(emb, feat, wc, wm, d0, d1, g0_o, g1_o, mlp_o, dinv_o):
    i = pl.program_id(0)
    dv = lax.rsqrt(d0[...] + d1[...] + 1.0)
    h = lax.dot_general(emb[...], wc[...], (((1,), (1,)), ((), ())),
                        preferred_element_type=jnp.float32)
    rid = i * _RB + lax.broadcasted_iota(jnp.int32, (_RB, 1), 0)
    g = jnp.where(rid < N, h * dv, 0.0).astype(jnp.bfloat16)
    g0_o[...] = g[:, :DH]
    g1_o[...] = g[:, DH:]
    mlp_o[...] = lax.dot_general(feat[...], wm[...], (((1,), (1,)), ((), ())),
                                 preferred_element_type=jnp.float32)
    dinv_o[...] = dv


_RB = 512


def _dense_call(emb, feat, conv_W, mlp_W, d0, d1):
    grid = (NPAD // _RB,)
    return pl.pallas_call(
        _dense_body,
        grid=grid,
        in_specs=[
            pl.BlockSpec((_RB, D), lambda i: (i, 0)),
            pl.BlockSpec((_RB, D), lambda i: (i, 0)),
            pl.BlockSpec((D, D), lambda i: (0, 0)),
            pl.BlockSpec((D, D), lambda i: (0, 0)),
            pl.BlockSpec((_RB, 1), lambda i: (i, 0)),
            pl.BlockSpec((_RB, 1), lambda i: (i, 0)),
        ],
        out_specs=[
            pl.BlockSpec((_RB, DH), lambda i: (i, 0)),
            pl.BlockSpec((_RB, DH), lambda i: (i, 0)),
            pl.BlockSpec((_RB, D), lambda i: (i, 0)),
            pl.BlockSpec((_RB, 1), lambda i: (i, 0)),
        ],
        out_shape=[
            jax.ShapeDtypeStruct((NPAD, DH), jnp.bfloat16),
            jax.ShapeDtypeStruct((NPAD, DH), jnp.bfloat16),
            jax.ShapeDtypeStruct((NPAD, D), jnp.float32),
            jax.ShapeDtypeStruct((NPAD, 1), jnp.float32),
        ],
    )(emb, feat, conv_W, mlp_W, d0, d1)


IBLK = 20
NPAIR = TCH // (2 * IBLK)


def _mp_body(src_hbm, dst_hbm, g0_hbm, g1_hbm, out_hbm, idxs0, idxd0, idxs1,
             idxd1, rows, ls0, ld0, ls1, ld1, gsems, acc, gtab):
    c = lax.axis_index("c")
    s = lax.axis_index("s")

    def _load_blk(k, idxs_b, idxd_b, sem_s, sem_d):
        base = s * TCH + k * IBLK
        pltpu.async_copy(src_hbm.at[pl.ds(base, IBLK)], idxs_b, sem_s)
        pltpu.async_copy(dst_hbm.at[pl.ds(base, IBLK)], idxd_b, sem_d)

    def _wait_blk(k, idxs_b, idxd_b, sem_s, sem_d):
        base = s * TCH + k * IBLK
        pltpu.make_async_copy(src_hbm.at[pl.ds(base, IBLK)], idxs_b,
                              sem_s).wait()
        pltpu.make_async_copy(dst_hbm.at[pl.ds(base, IBLK)], idxd_b,
                              sem_d).wait()

    _load_blk(0, idxs0, idxd0, ls0, ld0)

    @pl.when(c == 0)
    def _():
        pltpu.sync_copy(g0_hbm.at[pl.ds(s * RPW, RPW)],
                        gtab.at[pl.ds(s * RPW, RPW)])

    @pl.when(c == 1)
    def _():
        pltpu.sync_copy(g1_hbm.at[pl.ds(s * RPW, RPW)],
                        gtab.at[pl.ds(s * RPW, RPW)])

    def _zrow(r, _):
        for k in range(DH // 32):
            rows[0, r, pl.ds(k * 32, 32)] = jnp.zeros((32,), jnp.bfloat16)
        return _
    lax.fori_loop(0, CH, _zrow, None)
    for j in range(RPW // CH):
        pltpu.sync_copy(rows.at[0], acc.at[pl.ds(s * RPW + j * CH, CH)])
    plsc.subcore_barrier()

    def _block(idxs_b, idxd_b):
        for b in range(NBUF):
            pltpu.async_copy(gtab.at[idxs_b.at[b]], rows.at[b], gsems.at[b])

        def _chunk(i, _):
            for b in range(NBUF):
                j = i * NBUF + b
                pltpu.make_async_copy(gtab.at[idxs_b.at[j]], rows.at[b],
                                      gsems.at[b]).wait()
                pltpu.sync_copy(rows.at[b], acc.at[idxd_b.at[j]], add=True)
                pltpu.async_copy(gtab.at[idxs_b.at[j + NBUF]], rows.at[b],
                                 gsems.at[b])
            return _
        lax.fori_loop(0, (IBLK - NBUF) // NBUF, _chunk, None)

        for b in range(NBUF):
            j = IBLK - NBUF + b
            pltpu.make_async_copy(gtab.at[idxs_b.at[j]], rows.at[b],
                                  gsems.at[b]).wait()
            pltpu.sync_copy(rows.at[b], acc.at[idxd_b.at[j]], add=True)

    def _pair(i, _):
        k0 = 2 * i
        _wait_blk(k0, idxs0, idxd0, ls0, ld0)
        _load_blk(k0 + 1, idxs1, idxd1, ls1, ld1)
        _block(idxs0, idxd0)
        _wait_blk(k0 + 1, idxs1, idxd1, ls1, ld1)

        @pl.when(i < NPAIR - 1)
        def _():
            _load_blk(k0 + 2, idxs0, idxd0, ls0, ld0)

        _block(idxs1, idxd1)
        return _
    lax.fori_loop(0, NPAIR, _pair, None)

    plsc.subcore_barrier()

    for q in range(RPW // CH):
        r0 = s * RPW + q * CH
        pltpu.sync_copy(acc.at[pl.ds(r0, CH)], rows.at[0])
        pltpu.sync_copy(gtab.at[pl.ds(r0, CH)], rows.at[1])

        def _addrow(r, _):
            for k in range(DH // 32):
                sl = pl.ds(k * 32, 32)
                rows[0, r, sl] = rows[0, r, sl] + rows[1, r, sl]
            return _
        lax.fori_loop(0, CH, _addrow, None)
        pltpu.sync_copy(rows.at[0], out_hbm.at[c, pl.ds(r0, CH)])


_mp_call = functools.partial(
    pl.kernel,
    out_type=jax.ShapeDtypeStruct((NC, NPAD, DH), jnp.bfloat16),
    mesh=_mesh,
    scratch_types=[
        pltpu.VMEM((IBLK, CH), jnp.int32),
        pltpu.VMEM((IBLK, CH), jnp.int32),
        pltpu.VMEM((IBLK, CH), jnp.int32),
        pltpu.VMEM((IBLK, CH), jnp.int32),
        pltpu.VMEM((NBUF, CH, DH), jnp.bfloat16),
        pltpu.SemaphoreType.DMA,
        pltpu.SemaphoreType.DMA,
        pltpu.SemaphoreType.DMA,
        pltpu.SemaphoreType.DMA,
        pltpu.SemaphoreType.DMA((NBUF,)),
        pltpu.VMEM_SHARED((NPAD, DH), jnp.bfloat16),
        pltpu.VMEM_SHARED((NPAD, DH), jnp.bfloat16),
    ],
    compiler_params=_sc_params,
)(_mp_body)


def _comb_body(a0, a1, mlp, dinv, o):
    dv = dinv[...]
    m = mlp[...]
    f32 = jnp.float32
    left = a0[0].astype(f32) * dv + m[:, :DH]
    right = a1[0].astype(f32) * dv + m[:, DH:]
    o[...] = jnp.maximum(jnp.concatenate([left, right], axis=1), 0.0)


def _comb_call(accs, mlp, dinv):
    grid = (NPAD // _RB,)
    return pl.pallas_call(
        _comb_body,
        grid=grid,
        in_specs=[
            pl.BlockSpec((1, _RB, DH), lambda i: (0, i, 0)),
            pl.BlockSpec((1, _RB, DH), lambda i: (1, i, 0)),
            pl.BlockSpec((_RB, D), lambda i: (i, 0)),
            pl.BlockSpec((_RB, 1), lambda i: (i, 0)),
        ],
        out_specs=pl.BlockSpec((_RB, D), lambda i: (i, 0)),
        out_shape=jax.ShapeDtypeStruct((N, D), jnp.float32),
    )(accs, accs, mlp, dinv)


def kernel(feature, edge_index, embedding, conv_W, mlp_W):
    src = edge_index[0].astype(jnp.int32)
    dst = edge_index[1].astype(jnp.int32)
    pad = jnp.full((EPAD - E,), N, jnp.int32)
    src_p = jnp.concatenate([src, pad]).reshape(NS * TCH, CH)
    dst_p = jnp.concatenate([dst, pad]).reshape(NS * TCH, CH)

    deg2 = _deg_call(dst_p)
    d0 = deg2[0][:, None]
    d1 = deg2[1][:, None]
    g0, g1, mlp, dinv = _dense_call(embedding, feature, conv_W, mlp_W, d0, d1)
    accs = _mp_call(src_p, dst_p, g0, g1)
    return _comb_call(accs, mlp, dinv)


# speedup vs baseline: 40.6741x; 1.0693x over previous
import functools

import jax
import jax.numpy as jnp
from jax import lax
from jax.experimental import pallas as pl
from jax.experimental.pallas import tpu as pltpu
from jax.experimental.pallas import tpu_sc as plsc

N = 10000
E = 320000
D = 128
DH = D // 2
NC = 2
NS = 16
NW = NC * NS

NPAD = 10240
RPW = NPAD // NS
CH = 128
TCH = 160
EPAD = NS * TCH * CH
NBUF = 4

_mesh = plsc.VectorSubcoreMesh(
    core_axis_name="c", subcore_axis_name="s", num_cores=NC, num_subcores=NS)
_sc_params = pltpu.CompilerParams(needs_layout_passes=False,
                                  use_tc_tiling_on_sc=False,
                                  internal_scratch_in_bytes=128 * 1024)


def _deg_body(dst_hbm, deg_out, idxbuf, hist, vbuf, tot, shared):
    c = lax.axis_index("c")
    s = lax.axis_index("s")
    wid = s * NC + c
    npc = EPAD // NW // CH

    zero16 = jnp.zeros((16,), jnp.float32)

    def _zero(i, _):
        for u in range(8):
            hist[pl.ds(i * 128 + u * 16, 16)] = zero16
        return _
    lax.fori_loop(0, NPAD // 128, _zero, None)

    pltpu.sync_copy(dst_hbm.at[pl.ds(wid * npc, npc)], idxbuf)

    ones = jnp.ones((16,), jnp.float32)

    def _hist(t, _):
        for u in range(CH // 16):
            idx = idxbuf[t, pl.ds(u * 16, 16)]
            plsc.addupdate_scatter(hist, [idx], ones)
        return _
    lax.fori_loop(0, npc, _hist, None)

    pltpu.sync_copy(hist, shared.at[s])
    plsc.subcore_barrier()

    for r in range(NS):
        pltpu.sync_copy(shared.at[r, pl.ds(s * RPW, RPW)], vbuf.at[r])

    def _comb(v, _):
        a = vbuf[0, pl.ds(v * 16, 16)]
        for r in range(1, NS):
            a = a + vbuf[r, pl.ds(v * 16, 16)]
        tot[pl.ds(v * 16, 16)] = a
        return _
    lax.fori_loop(0, RPW // 16, _comb, None)

    pltpu.sync_copy(tot, deg_out.at[c, pl.ds(s * RPW, RPW)])


_deg_call = functools.partial(
    pl.kernel,
    out_type=jax.ShapeDtypeStruct((NC, NPAD), jnp.float32),
    mesh=_mesh,
    scratch_types=[
        pltpu.VMEM((EPAD // NW // CH, CH), jnp.int32),
        pltpu.VMEM((NPAD,), jnp.float32),
        pltpu.VMEM((NS, RPW), jnp.float32),
        pltpu.VMEM((RPW,), jnp.float32),
        pltpu.VMEM_SHARED((NS, NPAD), jnp.float32),
    ],
    compiler_params=_sc_params,
)(_deg_body)


def _dense_body(emb, feat, wc, wm, d0, d1, g0_o, g1_o, mlp_o, dinv_o):
    i = pl.program_id(0)
    dv = lax.rsqrt(d0[...] + d1[...] + 1.0)
    h = lax.dot_general(emb[...], wc[...], (((1,), (1,)), ((), ())),
                        preferred_element_type=jnp.float32)
    rid = i * _RB + lax.broadcasted_iota(jnp.int32, (_RB, 1), 0)
    g = jnp.where(rid < N, h * dv, 0.0).astype(jnp.bfloat16)
    g0_o[...] = g[:, :DH]
    g1_o[...] = g[:, DH:]
    mlp_o[...] = lax.dot_general(feat[...], wm[...], (((1,), (1,)), ((), ())),
                                 preferred_element_type=jnp.float32)
    dinv_o[...] = dv


_RB = 1024


def _dense_call(emb, feat, conv_W, mlp_W, d0, d1):
    grid = (NPAD // _RB,)
    return pl.pallas_call(
        _dense_body,
        grid=grid,
        in_specs=[
            pl.BlockSpec((_RB, D), lambda i: (i, 0)),
            pl.BlockSpec((_RB, D), lambda i: (i, 0)),
            pl.BlockSpec((D, D), lambda i: (0, 0)),
            pl.BlockSpec((D, D), lambda i: (0, 0)),
            pl.BlockSpec((_RB, 1), lambda i: (i, 0)),
            pl.BlockSpec((_RB, 1), lambda i: (i, 0)),
        ],
        out_specs=[
            pl.BlockSpec((_RB, DH), lambda i: (i, 0)),
            pl.BlockSpec((_RB, DH), lambda i: (i, 0)),
            pl.BlockSpec((_RB, D), lambda i: (i, 0)),
            pl.BlockSpec((_RB, 1), lambda i: (i, 0)),
        ],
        out_shape=[
            jax.ShapeDtypeStruct((NPAD, DH), jnp.bfloat16),
            jax.ShapeDtypeStruct((NPAD, DH), jnp.bfloat16),
            jax.ShapeDtypeStruct((NPAD, D), jnp.float32),
            jax.ShapeDtypeStruct((NPAD, 1), jnp.float32),
        ],
    )(emb, feat, conv_W, mlp_W, d0, d1)


IBLK = 20
NPAIR = TCH // (2 * IBLK)


def _mp_body(src_hbm, dst_hbm, g0_hbm, g1_hbm, out_hbm, idxs0, idxd0, idxs1,
             idxd1, rows, ls0, ld0, ls1, ld1, gsems, acc, gtab):
    c = lax.axis_index("c")
    s = lax.axis_index("s")

    def _load_blk(k, idxs_b, idxd_b, sem_s, sem_d):
        base = s * TCH + k * IBLK
        pltpu.async_copy(src_hbm.at[pl.ds(base, IBLK)], idxs_b, sem_s)
        pltpu.async_copy(dst_hbm.at[pl.ds(base, IBLK)], idxd_b, sem_d)

    def _wait_blk(k, idxs_b, idxd_b, sem_s, sem_d):
        base = s * TCH + k * IBLK
        pltpu.make_async_copy(src_hbm.at[pl.ds(base, IBLK)], idxs_b,
                              sem_s).wait()
        pltpu.make_async_copy(dst_hbm.at[pl.ds(base, IBLK)], idxd_b,
                              sem_d).wait()

    _load_blk(0, idxs0, idxd0, ls0, ld0)

    @pl.when(c == 0)
    def _():
        pltpu.sync_copy(g0_hbm.at[pl.ds(s * RPW, RPW)],
                        gtab.at[pl.ds(s * RPW, RPW)])

    @pl.when(c == 1)
    def _():
        pltpu.sync_copy(g1_hbm.at[pl.ds(s * RPW, RPW)],
                        gtab.at[pl.ds(s * RPW, RPW)])

    def _zrow(r, _):
        for k in range(DH // 32):
            rows[0, r, pl.ds(k * 32, 32)] = jnp.zeros((32,), jnp.bfloat16)
        return _
    lax.fori_loop(0, CH, _zrow, None)
    for j in range(RPW // CH):
        pltpu.sync_copy(rows.at[0], acc.at[pl.ds(s * RPW + j * CH, CH)])
    plsc.subcore_barrier()

    def _block(idxs_b, idxd_b):
        for b in range(NBUF):
            pltpu.async_copy(gtab.at[idxs_b.at[b]], rows.at[b], gsems.at[b])

        def _chunk(i, _):
            for b in range(NBUF):
                j = i * NBUF + b
                pltpu.make_async_copy(gtab.at[idxs_b.at[j]], rows.at[b],
                                      gsems.at[b]).wait()
                pltpu.sync_copy(rows.at[b], acc.at[idxd_b.at[j]], add=True)
                pltpu.async_copy(gtab.at[idxs_b.at[j + NBUF]], rows.at[b],
                                 gsems.at[b])
            return _
        lax.fori_loop(0, (IBLK - NBUF) // NBUF, _chunk, None)

        for b in range(NBUF):
            j = IBLK - NBUF + b
            pltpu.make_async_copy(gtab.at[idxs_b.at[j]], rows.at[b],
                                  gsems.at[b]).wait()
            pltpu.sync_copy(rows.at[b], acc.at[idxd_b.at[j]], add=True)

    def _pair(i, _):
        k0 = 2 * i
        _wait_blk(k0, idxs0, idxd0, ls0, ld0)
        _load_blk(k0 + 1, idxs1, idxd1, ls1, ld1)
        _block(idxs0, idxd0)
        _wait_blk(k0 + 1, idxs1, idxd1, ls1, ld1)

        @pl.when(i < NPAIR - 1)
        def _():
            _load_blk(k0 + 2, idxs0, idxd0, ls0, ld0)

        _block(idxs1, idxd1)
        return _
    lax.fori_loop(0, NPAIR, _pair, None)

    plsc.subcore_barrier()

    for q in range(RPW // CH):
        r0 = s * RPW + q * CH
        pltpu.sync_copy(acc.at[pl.ds(r0, CH)], rows.at[0])
        pltpu.sync_copy(gtab.at[pl.ds(r0, CH)], rows.at[1])

        def _addrow(r, _):
            for k in range(DH // 32):
                sl = pl.ds(k * 32, 32)
                rows[0, r, sl] = rows[0, r, sl] + rows[1, r, sl]
            return _
        lax.fori_loop(0, CH, _addrow, None)
        pltpu.sync_copy(rows.at[0], out_hbm.at[c, pl.ds(r0, CH)])


_mp_call = functools.partial(
    pl.kernel,
    out_type=jax.ShapeDtypeStruct((NC, NPAD, DH), jnp.bfloat16),
    mesh=_mesh,
    scratch_types=[
        pltpu.VMEM((IBLK, CH), jnp.int32),
        pltpu.VMEM((IBLK, CH), jnp.int32),
        pltpu.VMEM((IBLK, CH), jnp.int32),
        pltpu.VMEM((IBLK, CH), jnp.int32),
        pltpu.VMEM((NBUF, CH, DH), jnp.bfloat16),
        pltpu.SemaphoreType.DMA,
        pltpu.SemaphoreType.DMA,
        pltpu.SemaphoreType.DMA,
        pltpu.SemaphoreType.DMA,
        pltpu.SemaphoreType.DMA((NBUF,)),
        pltpu.VMEM_SHARED((NPAD, DH), jnp.bfloat16),
        pltpu.VMEM_SHARED((NPAD, DH), jnp.bfloat16),
    ],
    compiler_params=_sc_params,
)(_mp_body)


def _comb_body(a0, a1, mlp, dinv, o):
    dv = dinv[...]
    m = mlp[...]
    f32 = jnp.float32
    left = a0[0].astype(f32) * dv + m[:, :DH]
    right = a1[0].astype(f32) * dv + m[:, DH:]
    o[...] = jnp.maximum(jnp.concatenate([left, right], axis=1), 0.0)


def _comb_call(accs, mlp, dinv):
    grid = (NPAD // _RB,)
    return pl.pallas_call(
        _comb_body,
        grid=grid,
        in_specs=[
            pl.BlockSpec((1, _RB, DH), lambda i: (0, i, 0)),
            pl.BlockSpec((1, _RB, DH), lambda i: (1, i, 0)),
            pl.BlockSpec((_RB, D), lambda i: (i, 0)),
            pl.BlockSpec((_RB, 1), lambda i: (i, 0)),
        ],
        out_specs=pl.BlockSpec((_RB, D), lambda i: (i, 0)),
        out_shape=jax.ShapeDtypeStruct((N, D), jnp.float32),
    )(accs, accs, mlp, dinv)


def kernel(feature, edge_index, embedding, conv_W, mlp_W):
    src = edge_index[0].astype(jnp.int32)
    dst = edge_index[1].astype(jnp.int32)
    pad = jnp.full((EPAD - E,), N, jnp.int32)
    src_p = jnp.concatenate([src, pad]).reshape(NS * TCH, CH)
    dst_p = jnp.concatenate([dst, pad]).reshape(NS * TCH, CH)

    deg2 = _deg_call(dst_p)
    d0 = deg2[0][:, None]
    d1 = deg2[1][:, None]
    g0, g1, mlp, dinv = _dense_call(embedding, feature, conv_W, mlp_W, d0, d1)
    accs = _mp_call(src_p, dst_p, g0, g1)
    return _comb_call(accs, mlp, dinv)
